# baseline (speedup 1.0000x reference)
import jax
import jax.numpy as jnp
from jax.experimental import pallas as pl
from jax.experimental.pallas import tpu as pltpu

H = 32
W = 32


def _add_pos_kernel(r_ref, c_ref, row_tab_ref, col_tab_ref, out_in_ref,
                    out_ref, pos_ref):
    b = pl.program_id(0)

    @pl.when(b == 0)
    def _():
        s = r_ref.shape[0]
        row_oh = (jax.lax.broadcasted_iota(jnp.int32, (s, H), 1)
                  == r_ref[...]).astype(jnp.float32)
        col_oh = (jax.lax.broadcasted_iota(jnp.int32, (s, W), 1)
                  == c_ref[...]).astype(jnp.float32)
        pos_ref[...] = (
            jax.lax.dot(row_oh, row_tab_ref[...],
                        preferred_element_type=jnp.float32)
            + jax.lax.dot(col_oh, col_tab_ref[...],
                          preferred_element_type=jnp.float32))

    out_ref[...] = out_in_ref[...] + pos_ref[...]


def kernel(output, row_table, col_table, r, c):
    B, S, D = output.shape
    r2 = r.reshape(S, 1)
    c2 = c.reshape(S, 1)
    flat = output.reshape(B * S, D)
    res = pl.pallas_call(
        _add_pos_kernel,
        grid=(B,),
        in_specs=[
            pl.BlockSpec((S, 1), lambda b: (0, 0)),
            pl.BlockSpec((S, 1), lambda b: (0, 0)),
            pl.BlockSpec((H, D), lambda b: (0, 0)),
            pl.BlockSpec((W, D), lambda b: (0, 0)),
            pl.BlockSpec((S, D), lambda b: (b, 0)),
        ],
        out_specs=pl.BlockSpec((S, D), lambda b: (b, 0)),
        out_shape=jax.ShapeDtypeStruct((B * S, D), jnp.float32),
        scratch_shapes=[pltpu.VMEM((S, D), jnp.float32)],
    )(r2, c2, row_table, col_table, flat)
    return res.reshape(B, S, D)


# speedup vs baseline: 1.1926x; 1.1926x over previous
import jax
import jax.numpy as jnp
from jax.experimental import pallas as pl
from jax.experimental.pallas import tpu as pltpu

H = 32
W = 32


def _add_pos_kernel(r_ref, c_ref, row_tab_ref, col_tab_ref, out_in_ref,
                    out_ref, pos_ref):
    b = pl.program_id(0)

    @pl.when(b == 0)
    def _():
        s = r_ref.shape[0]
        row_oh = (jax.lax.broadcasted_iota(jnp.int32, (s, H), 1)
                  == r_ref[...]).astype(jnp.float32)
        col_oh = (jax.lax.broadcasted_iota(jnp.int32, (s, W), 1)
                  == c_ref[...]).astype(jnp.float32)
        pos_ref[...] = (
            jax.lax.dot(row_oh, row_tab_ref[...],
                        preferred_element_type=jnp.float32)
            + jax.lax.dot(col_oh, col_tab_ref[...],
                          preferred_element_type=jnp.float32))

    s = pos_ref.shape[0]
    nrep = out_ref.shape[0] // s
    for i in range(nrep):
        out_ref[i * s:(i + 1) * s, :] = (
            out_in_ref[i * s:(i + 1) * s, :] + pos_ref[...])


_BATCHES_PER_BLOCK = 2


def kernel(output, row_table, col_table, r, c):
    B, S, D = output.shape
    r2 = r.reshape(S, 1)
    c2 = c.reshape(S, 1)
    flat = output.reshape(B * S, D)
    nb = _BATCHES_PER_BLOCK
    rows = nb * S
    res = pl.pallas_call(
        _add_pos_kernel,
        grid=(B // nb,),
        in_specs=[
            pl.BlockSpec((S, 1), lambda b: (0, 0)),
            pl.BlockSpec((S, 1), lambda b: (0, 0)),
            pl.BlockSpec((H, D), lambda b: (0, 0)),
            pl.BlockSpec((W, D), lambda b: (0, 0)),
            pl.BlockSpec((rows, D), lambda b: (b, 0)),
        ],
        out_specs=pl.BlockSpec((rows, D), lambda b: (b, 0)),
        out_shape=jax.ShapeDtypeStruct((B * S, D), jnp.float32),
        scratch_shapes=[pltpu.VMEM((S, D), jnp.float32)],
    )(r2, c2, row_table, col_table, flat)
    return res.reshape(B, S, D)


# speedup vs baseline: 1.2396x; 1.0394x over previous
import jax
import jax.numpy as jnp
from jax.experimental import pallas as pl
from jax.experimental.pallas import tpu as pltpu

H = 32
W = 32


def _add_pos_kernel(r_ref, c_ref, row_tab_ref, col_tab_ref, out_in_ref,
                    out_ref, pos_ref):
    b = pl.program_id(0)

    @pl.when(b == 0)
    def _():
        s = r_ref.shape[0]
        row_oh = (jax.lax.broadcasted_iota(jnp.int32, (s, H), 1)
                  == r_ref[...]).astype(jnp.float32)
        col_oh = (jax.lax.broadcasted_iota(jnp.int32, (s, W), 1)
                  == c_ref[...]).astype(jnp.float32)
        pos_ref[...] = (
            jax.lax.dot(row_oh, row_tab_ref[...],
                        preferred_element_type=jnp.float32)
            + jax.lax.dot(col_oh, col_tab_ref[...],
                          preferred_element_type=jnp.float32))

    s = pos_ref.shape[0]
    nrep = out_ref.shape[0] // s
    for i in range(nrep):
        out_ref[i * s:(i + 1) * s, :] = (
            out_in_ref[i * s:(i + 1) * s, :] + pos_ref[...])


_BATCHES_PER_BLOCK = 4


def kernel(output, row_table, col_table, r, c):
    B, S, D = output.shape
    r2 = r.reshape(S, 1)
    c2 = c.reshape(S, 1)
    flat = output.reshape(B * S, D)
    nb = _BATCHES_PER_BLOCK
    rows = nb * S
    res = pl.pallas_call(
        _add_pos_kernel,
        grid=(B // nb,),
        in_specs=[
            pl.BlockSpec((S, 1), lambda b: (0, 0)),
            pl.BlockSpec((S, 1), lambda b: (0, 0)),
            pl.BlockSpec((H, D), lambda b: (0, 0)),
            pl.BlockSpec((W, D), lambda b: (0, 0)),
            pl.BlockSpec((rows, D), lambda b: (b, 0)),
        ],
        out_specs=pl.BlockSpec((rows, D), lambda b: (b, 0)),
        out_shape=jax.ShapeDtypeStruct((B * S, D), jnp.float32),
        scratch_shapes=[pltpu.VMEM((S, D), jnp.float32)],
    )(r2, c2, row_table, col_table, flat)
    return res.reshape(B, S, D)


# speedup vs baseline: 1.2842x; 1.0360x over previous
import jax
import jax.numpy as jnp
from jax.experimental import pallas as pl
from jax.experimental.pallas import tpu as pltpu

H = 32
W = 32


def _add_pos_kernel(r_ref, c_ref, row_tab_ref, col_tab_ref, out_in_ref,
                    out_ref, pos_ref):
    b = pl.program_id(0)

    @pl.when(b == 0)
    def _():
        s = r_ref.shape[0]
        row_oh = (jax.lax.broadcasted_iota(jnp.int32, (s, H), 1)
                  == r_ref[...]).astype(jnp.float32)
        col_oh = (jax.lax.broadcasted_iota(jnp.int32, (s, W), 1)
                  == c_ref[...]).astype(jnp.float32)
        pos_ref[...] = (
            jax.lax.dot(row_oh, row_tab_ref[...],
                        preferred_element_type=jnp.float32)
            + jax.lax.dot(col_oh, col_tab_ref[...],
                          preferred_element_type=jnp.float32))

    s = pos_ref.shape[0]
    nrep = out_ref.shape[0] // s
    for i in range(nrep):
        out_ref[i * s:(i + 1) * s, :] = (
            out_in_ref[i * s:(i + 1) * s, :] + pos_ref[...])


_BATCHES_PER_BLOCK = 8


def kernel(output, row_table, col_table, r, c):
    B, S, D = output.shape
    r2 = r.reshape(S, 1)
    c2 = c.reshape(S, 1)
    flat = output.reshape(B * S, D)
    nb = _BATCHES_PER_BLOCK
    rows = nb * S
    res = pl.pallas_call(
        _add_pos_kernel,
        grid=(B // nb,),
        in_specs=[
            pl.BlockSpec((S, 1), lambda b: (0, 0)),
            pl.BlockSpec((S, 1), lambda b: (0, 0)),
            pl.BlockSpec((H, D), lambda b: (0, 0)),
            pl.BlockSpec((W, D), lambda b: (0, 0)),
            pl.BlockSpec((rows, D), lambda b: (b, 0)),
        ],
        out_specs=pl.BlockSpec((rows, D), lambda b: (b, 0)),
        out_shape=jax.ShapeDtypeStruct((B * S, D), jnp.float32),
        scratch_shapes=[pltpu.VMEM((S, D), jnp.float32)],
    )(r2, c2, row_table, col_table, flat)
    return res.reshape(B, S, D)
